# baseline (speedup 1.0000x reference)
import jax
import jax.numpy as jnp
from jax.experimental import pallas as pl
from jax.experimental.pallas import tpu as pltpu

B, S, D, H, O, E, K = 1, 2048, 1024, 1024, 1024, 8, 2
TB = 128
NE = E + 1


def _dot(a, b, precision=None):
    return jax.lax.dot_general(
        a, b, (((1,), (0,)), ((), ())),
        precision=precision, preferred_element_type=jnp.float32)


def _moe_body(x_ref, nw_ref, rwt_ref, w1c_ref, b1c_ref, w2s_ref, b2e_ref,
              sh2b_ref, o_ref):
    for t in range(S // TB):
        x = x_ref[pl.ds(t * TB, TB), :]
        var = jnp.mean(x * x, axis=-1, keepdims=True)
        normed = x * jax.lax.rsqrt(var + 1e-8) * nw_ref[...]

        logits = _dot(normed, rwt_ref[...])
        probs = 1.0 / (1.0 + jnp.exp(-logits))
        eidx = jax.lax.broadcasted_iota(jnp.int32, probs.shape, 1)
        m1 = jnp.max(probs, axis=-1, keepdims=True)
        i1 = jnp.min(jnp.where(probs == m1, eidx, E), axis=-1, keepdims=True)
        probs2 = jnp.where(eidx == i1, -1.0, probs)
        m2 = jnp.max(probs2, axis=-1, keepdims=True)
        i2 = jnp.min(jnp.where(probs2 == m2, eidx, E), axis=-1, keepdims=True)
        denom = m1 + m2 + 1e-6
        cw = (jnp.where(eidx == i1, m1, 0.0)
              + jnp.where(eidx == i2, m2, 0.0)) / denom

        nb = normed.astype(jnp.bfloat16)
        pieces = []
        for e in range(NE):
            h = jnp.maximum(
                _dot(nb, w1c_ref[:, pl.ds(e * H, H)])
                + b1c_ref[:, pl.ds(e * H, H)], 0.0)
            if e > 0:
                h = h * cw[:, e - 1:e]
            pieces.append(h.astype(jnp.bfloat16))
        hs = jnp.concatenate(pieces, axis=1)
        acc = _dot(hs, w2s_ref[...]) + sh2b_ref[...] + _dot(cw, b2e_ref[...])
        o_ref[pl.ds(t * TB, TB), :] = acc


def kernel(hidden_states, norm_w, router_w, sh_fc1_w, sh_fc1_b, sh_fc2_w,
           sh_fc2_b, ex_fc1_w, ex_fc1_b, ex_fc2_w, ex_fc2_b):
    x = hidden_states.reshape(S, D)
    rwt = router_w.T
    w1c = jnp.concatenate(
        [sh_fc1_w.T.reshape(D, 1, H),
         ex_fc1_w.transpose(2, 0, 1)], axis=1).reshape(D, NE * H)
    w1c = w1c.astype(jnp.bfloat16)
    b1c = jnp.concatenate(
        [sh_fc1_b.reshape(1, H), ex_fc1_b.reshape(1, E * H)],
        axis=1)
    w2s = jnp.concatenate(
        [sh_fc2_w.T.reshape(1, H, O),
         ex_fc2_w.transpose(0, 2, 1)], axis=0).reshape(NE * H, O)
    w2s = w2s.astype(jnp.bfloat16)

    out = pl.pallas_call(
        _moe_body,
        grid=(1,),
        in_specs=[
            pl.BlockSpec((S, D), lambda i: (0, 0)),
            pl.BlockSpec((1, D), lambda i: (0, 0)),
            pl.BlockSpec((D, E), lambda i: (0, 0)),
            pl.BlockSpec((D, NE * H), lambda i: (0, 0)),
            pl.BlockSpec((1, NE * H), lambda i: (0, 0)),
            pl.BlockSpec((NE * H, O), lambda i: (0, 0)),
            pl.BlockSpec((E, O), lambda i: (0, 0)),
            pl.BlockSpec((1, O), lambda i: (0, 0)),
        ],
        out_specs=pl.BlockSpec((S, O), lambda i: (0, 0)),
        out_shape=jax.ShapeDtypeStruct((S, O), jnp.float32),
        compiler_params=pltpu.CompilerParams(
            dimension_semantics=("arbitrary",),
        ),
    )(x, norm_w.reshape(1, D), rwt, w1c, b1c, w2s, ex_fc2_b,
      sh_fc2_b.reshape(1, O))
    return out.reshape(B, S, O)


# speedup vs baseline: 1.0434x; 1.0434x over previous
import jax
import jax.numpy as jnp
from jax.experimental import pallas as pl
from jax.experimental.pallas import tpu as pltpu

B, S, D, H, O, E, K = 1, 2048, 1024, 1024, 1024, 8, 2
TB = 256
NE = E + 1


def _dot(a, b, precision=None):
    return jax.lax.dot_general(
        a, b, (((1,), (0,)), ((), ())),
        precision=precision, preferred_element_type=jnp.float32)


def _moe_body(x_ref, nw_ref, rwt_ref, w1c_ref, b1c_ref, w2s_ref, b2e_ref,
              sh2b_ref, o_ref):
    x = x_ref[...]
    var = jnp.mean(x * x, axis=-1, keepdims=True)
    normed = x * jax.lax.rsqrt(var + 1e-8) * nw_ref[...]

    logits = _dot(normed, rwt_ref[...])
    probs = 1.0 / (1.0 + jnp.exp(-logits))
    eidx = jax.lax.broadcasted_iota(jnp.int32, probs.shape, 1)
    m1 = jnp.max(probs, axis=-1, keepdims=True)
    i1 = jnp.min(jnp.where(probs == m1, eidx, E), axis=-1, keepdims=True)
    probs2 = jnp.where(eidx == i1, -1.0, probs)
    m2 = jnp.max(probs2, axis=-1, keepdims=True)
    i2 = jnp.min(jnp.where(probs2 == m2, eidx, E), axis=-1, keepdims=True)
    denom = m1 + m2 + 1e-6
    cw = (jnp.where(eidx == i1, m1, 0.0)
          + jnp.where(eidx == i2, m2, 0.0)) / denom

    nb = normed.astype(jnp.bfloat16)
    pieces = []
    for e in range(NE):
        h = jnp.maximum(
            _dot(nb, w1c_ref[:, pl.ds(e * H, H)])
            + b1c_ref[:, pl.ds(e * H, H)], 0.0)
        if e > 0:
            h = h * cw[:, e - 1:e]
        pieces.append(h.astype(jnp.bfloat16))
    hs = jnp.concatenate(pieces, axis=1)
    acc = _dot(hs, w2s_ref[...]) + sh2b_ref[...] + _dot(cw, b2e_ref[...])
    o_ref[...] = acc


def kernel(hidden_states, norm_w, router_w, sh_fc1_w, sh_fc1_b, sh_fc2_w,
           sh_fc2_b, ex_fc1_w, ex_fc1_b, ex_fc2_w, ex_fc2_b):
    x = hidden_states.reshape(S, D)
    rwt = router_w.T
    w1c = jnp.concatenate(
        [sh_fc1_w.T.reshape(D, 1, H),
         ex_fc1_w.transpose(2, 0, 1)], axis=1).reshape(D, NE * H)
    w1c = w1c.astype(jnp.bfloat16)
    b1c = jnp.concatenate(
        [sh_fc1_b.reshape(1, H), ex_fc1_b.reshape(1, E * H)],
        axis=1)
    w2s = jnp.concatenate(
        [sh_fc2_w.T.reshape(1, H, O),
         ex_fc2_w.transpose(0, 2, 1)], axis=0).reshape(NE * H, O)
    w2s = w2s.astype(jnp.bfloat16)

    tok = lambda i: (i, 0)
    whole = lambda i: (0, 0)
    out = pl.pallas_call(
        _moe_body,
        grid=(S // TB,),
        in_specs=[
            pl.BlockSpec((TB, D), tok),
            pl.BlockSpec((1, D), whole),
            pl.BlockSpec((D, E), whole),
            pl.BlockSpec((D, NE * H), whole),
            pl.BlockSpec((1, NE * H), whole),
            pl.BlockSpec((NE * H, O), whole),
            pl.BlockSpec((E, O), whole),
            pl.BlockSpec((1, O), whole),
        ],
        out_specs=pl.BlockSpec((TB, O), tok),
        out_shape=jax.ShapeDtypeStruct((S, O), jnp.float32),
        compiler_params=pltpu.CompilerParams(
            dimension_semantics=("arbitrary",),
        ),
    )(x, norm_w.reshape(1, D), rwt, w1c, b1c, w2s, ex_fc2_b,
      sh_fc2_b.reshape(1, O))
    return out.reshape(B, S, O)


# speedup vs baseline: 1.2723x; 1.2194x over previous
import functools

import jax
import jax.numpy as jnp
from jax.experimental import pallas as pl
from jax.experimental.pallas import tpu as pltpu

B, S, D, H, O, E, K = 1, 2048, 1024, 1024, 1024, 8, 2
TB = 512


def _dot(a, b, precision=None):
    return jax.lax.dot_general(
        a, b, (((1,), (0,)), ((), ())),
        precision=precision, preferred_element_type=jnp.float32)


def _moe_body(x_ref, nw_ref, rwt_ref, sh1t_ref, sh1b_ref, sh2t_ref, sh2b_ref,
              w1t_ref, b1_ref, w2t_ref, b2_ref, o_ref):
    x = x_ref[...]
    var = jnp.mean(x * x, axis=-1, keepdims=True)
    normed = x * jax.lax.rsqrt(var + 1e-8) * nw_ref[...]

    logits = _dot(normed, rwt_ref[...])
    probs = 1.0 / (1.0 + jnp.exp(-logits))
    eidx = jax.lax.broadcasted_iota(jnp.int32, probs.shape, 1)
    m1 = jnp.max(probs, axis=-1, keepdims=True)
    i1 = jnp.min(jnp.where(probs == m1, eidx, E), axis=-1, keepdims=True)
    probs2 = jnp.where(eidx == i1, -1.0, probs)
    m2 = jnp.max(probs2, axis=-1, keepdims=True)
    i2 = jnp.min(jnp.where(probs2 == m2, eidx, E), axis=-1, keepdims=True)
    denom = m1 + m2 + 1e-6
    cw = (jnp.where(eidx == i1, m1, 0.0) + jnp.where(eidx == i2, m2, 0.0)) / denom

    nb = normed.astype(jnp.bfloat16)
    h = jnp.maximum(_dot(nb, sh1t_ref[...]) + sh1b_ref[...], 0.0)
    acc = _dot(h.astype(jnp.bfloat16), sh2t_ref[...]) + sh2b_ref[...]
    for e in range(E):
        he = jnp.maximum(_dot(nb, w1t_ref[e]) + b1_ref[e], 0.0)
        ye = _dot(he.astype(jnp.bfloat16), w2t_ref[e]) + b2_ref[e]
        acc = acc + cw[:, e:e + 1] * ye
    o_ref[...] = acc


def kernel(hidden_states, norm_w, router_w, sh_fc1_w, sh_fc1_b, sh_fc2_w,
           sh_fc2_b, ex_fc1_w, ex_fc1_b, ex_fc2_w, ex_fc2_b):
    x = hidden_states.reshape(S, D)
    rwt = router_w.T
    sh1t = sh_fc1_w.T.astype(jnp.bfloat16)
    sh2t = sh_fc2_w.T.astype(jnp.bfloat16)
    w1t = ex_fc1_w.transpose(0, 2, 1).astype(jnp.bfloat16)
    w2t = ex_fc2_w.transpose(0, 2, 1).astype(jnp.bfloat16)

    grid = (S // TB,)
    tok = lambda i: (i, 0)
    whole2 = lambda i: (0, 0)
    whole3 = lambda i: (0, 0, 0)
    out = pl.pallas_call(
        _moe_body,
        grid=grid,
        in_specs=[
            pl.BlockSpec((TB, D), tok),
            pl.BlockSpec((1, D), whole2),
            pl.BlockSpec((D, E), whole2),
            pl.BlockSpec((D, H), whole2),
            pl.BlockSpec((1, H), whole2),
            pl.BlockSpec((H, O), whole2),
            pl.BlockSpec((1, O), whole2),
            pl.BlockSpec((E, D, H), whole3),
            pl.BlockSpec((E, H), whole2),
            pl.BlockSpec((E, H, O), whole3),
            pl.BlockSpec((E, O), whole2),
        ],
        out_specs=pl.BlockSpec((TB, O), tok),
        out_shape=jax.ShapeDtypeStruct((S, O), jnp.float32),
        compiler_params=pltpu.CompilerParams(
            dimension_semantics=("arbitrary",),
        ),
    )(x, norm_w.reshape(1, D), rwt, sh1t, sh_fc1_b.reshape(1, H), sh2t,
      sh_fc2_b.reshape(1, O), w1t, ex_fc1_b, w2t, ex_fc2_b)
    return out.reshape(B, S, O)
